# baseline (speedup 1.0000x reference)
import functools

import jax
import jax.numpy as jnp
from jax import lax
from jax.experimental import pallas as pl
from jax.experimental.pallas import tpu as pltpu
from jax.experimental.pallas import tpu_sc as plsc

_LANE = 16
_ROW = 128
_LAG = 8


@functools.cache
def _build(B, V, C):
    info = plsc.get_sparse_core_info()
    nc, ns = info.num_cores, info.num_subcores
    nw = nc * ns
    b_per_w = B // nw
    n_rows = (b_per_w * C) // _ROW
    rows_per_c = b_per_w // _ROW
    trow_span = ((V + 127) // 128) * 1024

    mesh = plsc.VectorSubcoreMesh(core_axis_name="c", subcore_axis_name="s")

    @functools.partial(
        pl.kernel,
        mesh=mesh,
        out_type=jax.ShapeDtypeStruct((C, B), jnp.float32),
        scratch_types=[
            pltpu.VMEM((b_per_w,), jnp.int32),
            pltpu.VMEM((b_per_w,), jnp.int32),
            pltpu.VMEM((n_rows, _ROW), jnp.int32),
            pltpu.VMEM((C, b_per_w), jnp.float32),
            pltpu.SemaphoreType.DMA,
        ],
        compiler_params=pltpu.CompilerParams(use_tc_tiling_on_sc=False),
    )
    def gather_kernel(lab_hbm, tab_hbm, out_hbm, lab_v, off_v, idx_v, dst_v, sem):
        wid = lax.axis_index("s") * nc + lax.axis_index("c")
        base = wid * b_per_w
        tab_row = tab_hbm.at[0]

        pltpu.sync_copy(lab_hbm.at[pl.ds(base, b_per_w)], lab_v)

        @pl.loop(0, n_rows)
        def build_row(r):
            c = r // rows_per_c
            cbase = c * V
            col = (r % rows_per_c) * _ROW
            for k in range(_ROW // _LANE):
                idx_v[r, pl.ds(k * _LANE, _LANE)] = (
                    lab_v[pl.ds(col + k * _LANE, _LANE)] + cbase
                )

        def dst_row(r):
            c = r // rows_per_c
            col = (r % rows_per_c) * _ROW
            return dst_v.at[c, pl.ds(col, _ROW)]

        @pl.loop(0, _LAG)
        def prime(r):
            pltpu.async_copy(tab_row.at[idx_v.at[r]], dst_row(r), sem)

        @pl.loop(_LAG, n_rows)
        def steady(r):
            pltpu.async_copy(tab_row.at[idx_v.at[r]], dst_row(r), sem)
            pltpu.make_async_copy(tab_row.at[idx_v.at[0]], dst_row(0), sem).wait()

        @pl.loop(0, _LAG)
        def tail(r):
            pltpu.make_async_copy(tab_row.at[idx_v.at[0]], dst_row(0), sem).wait()

        pltpu.sync_copy(dst_v, out_hbm.at[:, pl.ds(base, b_per_w)])

    def run(lab, tab_t):
        out_t = gather_kernel(lab, tab_t)
        return out_t.T.reshape(B, 1, C)

    return run


def kernel(label, table):
    B = label.shape[0]
    V, C = table.shape
    lab = label.reshape(-1).astype(jnp.int32)
    return _build(B, V, C)(lab, table.T)


# speedup vs baseline: 8.1019x; 8.1019x over previous
import functools

import jax
import jax.numpy as jnp
from jax import lax
from jax.experimental import pallas as pl
from jax.experimental.pallas import tpu as pltpu
from jax.experimental.pallas import tpu_sc as plsc

_CHUNK = 128


@functools.cache
def _build(B, V, C):
    info = plsc.get_sparse_core_info()
    nc, ns = info.num_cores, info.num_subcores
    nw = nc * ns
    b_per_w = B // nw
    n_chunks = b_per_w // _CHUNK

    mesh = plsc.VectorSubcoreMesh(core_axis_name="c", subcore_axis_name="s")

    @functools.partial(
        pl.kernel,
        mesh=mesh,
        out_type=jax.ShapeDtypeStruct((nw, n_chunks, _CHUNK, C), jnp.float32),
        scratch_types=[
            pltpu.VMEM((n_chunks, _CHUNK), jnp.int32),
            pltpu.VMEM((n_chunks, _CHUNK, C), jnp.float32),
            pltpu.SemaphoreType.DMA,
        ],
        compiler_params=pltpu.CompilerParams(use_tc_tiling_on_sc=False),
    )
    def gather_kernel(idx_hbm, table_hbm, out_hbm, idx_v, rows_v, sem):
        wid = lax.axis_index("s") * nc + lax.axis_index("c")
        pltpu.sync_copy(idx_hbm.at[wid], idx_v)
        copies = [
            pltpu.async_copy(table_hbm.at[idx_v.at[j]], rows_v.at[j], sem)
            for j in range(n_chunks)
        ]
        for c in copies:
            c.wait()
        pltpu.sync_copy(rows_v, out_hbm.at[wid])

    def run(idx, table):
        idx3 = idx.reshape(nw, n_chunks, _CHUNK)
        out = gather_kernel(idx3, table)
        return out.reshape(B, 1, C)

    return run


def kernel(label, table):
    B = label.shape[0]
    V, C = table.shape
    idx = label.reshape(-1).astype(jnp.int32)
    return _build(B, V, C)(idx, table)
